# Initial kernel scaffold; baseline (speedup 1.0000x reference)
import jax
import jax.numpy as jnp
from jax.experimental import pallas as pl


def kernel(hidden_states, gate_w, Wg, Wu, Wd, step_num):
    raise NotImplementedError("write your pallas kernel here")


# speedup vs baseline: 1.2566x; 1.2566x over previous
import functools

import jax
import jax.numpy as jnp
from jax import lax
from jax.experimental import pallas as pl
from jax.experimental.pallas import tpu as pltpu

E = 8
TOP_K = 2
D = 1024
F = 2048
TILE = 256
NT = (E * TOP_K * TILE + 0) // TILE

S_TOK = 2048
NT = (S_TOK * TOP_K) // TILE + E
NP = NT * TILE


def _topk2(logits):
    n = logits.shape[-1]
    idx = jnp.arange(n, dtype=jnp.float32)
    composite = -logits.astype(jnp.float32) + idx * 1e-06
    _, topk_idx = jax.lax.top_k(-composite, TOP_K)
    topk_vals = jnp.take_along_axis(logits, topk_idx, axis=-1)
    return topk_vals, topk_idx


def _expert_mlp_body(e_ref, v_ref, x_ref, wg_ref, wu_ref, wd_ref, w_ref, o_ref):
    i = pl.program_id(0)

    @pl.when(v_ref[i] == 1)
    def _():
        x = x_ref[...]
        wg = wg_ref[0].astype(jnp.bfloat16)
        g = lax.dot_general(x, wg, (((1,), (1,)), ((), ())),
                            preferred_element_type=jnp.float32)
        wu = wu_ref[0].astype(jnp.bfloat16)
        u = lax.dot_general(x, wu, (((1,), (1,)), ((), ())),
                            preferred_element_type=jnp.float32)
        h = g * jax.nn.sigmoid(g) * u
        wd = wd_ref[0].astype(jnp.bfloat16)
        out = lax.dot_general(h.astype(jnp.bfloat16), wd,
                              (((1,), (1,)), ((), ())),
                              preferred_element_type=jnp.float32)
        o_ref[...] = out * w_ref[...]


def _grouped_mlp(x_sorted, Wg, Wu, Wd, w_col, tile_expert, valid):
    grid_spec = pltpu.PrefetchScalarGridSpec(
        num_scalar_prefetch=2,
        grid=(NT,),
        in_specs=[
            pl.BlockSpec((TILE, D), lambda i, e, v: (i, 0)),
            pl.BlockSpec((1, F, D), lambda i, e, v: (e[i], 0, 0)),
            pl.BlockSpec((1, F, D), lambda i, e, v: (e[i], 0, 0)),
            pl.BlockSpec((1, D, F), lambda i, e, v: (e[i], 0, 0)),
            pl.BlockSpec((TILE, 1), lambda i, e, v: (i, 0)),
        ],
        out_specs=pl.BlockSpec((TILE, D), lambda i, e, v: (i, 0)),
    )
    return pl.pallas_call(
        _expert_mlp_body,
        grid_spec=grid_spec,
        out_shape=jax.ShapeDtypeStruct((NP, D), jnp.float32),
    )(tile_expert, valid, x_sorted, Wg, Wu, Wd, w_col)


def kernel(hidden_states, gate_w, Wg, Wu, Wd, step_num):
    b, s, d = hidden_states.shape
    x = hidden_states.reshape(-1, d)

    router_logits = x @ gate_w.T
    routing_vals, selected_experts = _topk2(router_logits)
    routing_weights = jax.nn.softmax(routing_vals.astype(jnp.float32), axis=-1)

    ef = selected_experts.reshape(-1).astype(jnp.int32)
    wf = routing_weights.reshape(-1)
    sort_idx = jnp.argsort(ef)
    ef_s = ef[sort_idx]
    counts = jnp.zeros((E,), jnp.int32).at[ef].add(1)
    offsets = jnp.concatenate([jnp.zeros((1,), jnp.int32),
                               jnp.cumsum(counts)[:-1]])
    ntiles = (counts + TILE - 1) // TILE
    tile_ofs = jnp.concatenate([jnp.zeros((1,), jnp.int32),
                                jnp.cumsum(ntiles)[:-1]])
    nslots = ef.shape[0]
    rank = jnp.arange(nslots, dtype=jnp.int32) - offsets[ef_s]
    pos = tile_ofs[ef_s] * TILE + rank

    t_padded = jnp.zeros((NP,), jnp.int32).at[pos].set(
        (sort_idx // TOP_K).astype(jnp.int32))
    w_padded = jnp.zeros((NP,), jnp.float32).at[pos].set(wf[sort_idx])
    tile_expert = jnp.repeat(jnp.arange(E, dtype=jnp.int32), ntiles,
                             total_repeat_length=NT)
    n_real = jnp.sum(ntiles)
    valid = (jnp.arange(NT, dtype=jnp.int32) < n_real).astype(jnp.int32)

    inv_pos = jnp.zeros((nslots,), jnp.int32).at[sort_idx].set(pos)
    p0 = inv_pos[0::TOP_K]
    p1 = inv_pos[1::TOP_K]

    x_bf = x.astype(jnp.bfloat16)
    x_sorted = x_bf[t_padded]
    out_sorted = _grouped_mlp(x_sorted, Wg, Wu, Wd, w_padded[:, None],
                              tile_expert, valid)
    final = out_sorted[p0] + out_sorted[p1]
    return final.reshape(b, s, d)


# speedup vs baseline: 1.4125x; 1.1240x over previous
import functools

import jax
import jax.numpy as jnp
from jax import lax
from jax.experimental import pallas as pl
from jax.experimental.pallas import tpu as pltpu

E = 8
TOP_K = 2
D = 1024
F = 2048
TILE = 256
NT = (E * TOP_K * TILE + 0) // TILE

S_TOK = 2048
NT = (S_TOK * TOP_K) // TILE + E
NP = NT * TILE


def _topk2(logits):
    n = logits.shape[-1]
    idx = jnp.arange(n, dtype=jnp.float32)
    composite = -logits.astype(jnp.float32) + idx * 1e-06
    iidx = jnp.arange(n, dtype=jnp.int32)
    big = jnp.int32(n)
    c1 = jnp.min(composite, axis=-1, keepdims=True)
    i1 = jnp.min(jnp.where(composite == c1, iidx, big), axis=-1, keepdims=True)
    masked = jnp.where(iidx == i1, jnp.inf, composite)
    c2 = jnp.min(masked, axis=-1, keepdims=True)
    i2 = jnp.min(jnp.where(masked == c2, iidx, big), axis=-1, keepdims=True)
    topk_idx = jnp.concatenate([i1, i2], axis=-1)
    topk_vals = jnp.take_along_axis(logits, topk_idx, axis=-1)
    return topk_vals, topk_idx


def _expert_mlp_body(e_ref, v_ref, x_ref, wg_ref, wu_ref, wd_ref, w_ref, o_ref):
    i = pl.program_id(0)

    @pl.when(v_ref[i] == 1)
    def _():
        x = x_ref[...]
        wg = wg_ref[0].astype(jnp.bfloat16)
        g = lax.dot_general(x, wg, (((1,), (1,)), ((), ())),
                            preferred_element_type=jnp.float32)
        wu = wu_ref[0].astype(jnp.bfloat16)
        u = lax.dot_general(x, wu, (((1,), (1,)), ((), ())),
                            preferred_element_type=jnp.float32)
        h = g * jax.nn.sigmoid(g) * u
        wd = wd_ref[0].astype(jnp.bfloat16)
        out = lax.dot_general(h.astype(jnp.bfloat16), wd,
                              (((1,), (1,)), ((), ())),
                              preferred_element_type=jnp.float32)
        o_ref[...] = out * w_ref[...]


def _grouped_mlp(x_sorted, Wg, Wu, Wd, w_col, tile_expert, valid):
    grid_spec = pltpu.PrefetchScalarGridSpec(
        num_scalar_prefetch=2,
        grid=(NT,),
        in_specs=[
            pl.BlockSpec((TILE, D), lambda i, e, v: (i, 0)),
            pl.BlockSpec((1, F, D), lambda i, e, v: (e[i], 0, 0)),
            pl.BlockSpec((1, F, D), lambda i, e, v: (e[i], 0, 0)),
            pl.BlockSpec((1, D, F), lambda i, e, v: (e[i], 0, 0)),
            pl.BlockSpec((TILE, 1), lambda i, e, v: (i, 0)),
        ],
        out_specs=pl.BlockSpec((TILE, D), lambda i, e, v: (i, 0)),
    )
    return pl.pallas_call(
        _expert_mlp_body,
        grid_spec=grid_spec,
        out_shape=jax.ShapeDtypeStruct((NP, D), jnp.float32),
    )(tile_expert, valid, x_sorted, Wg, Wu, Wd, w_col)


def kernel(hidden_states, gate_w, Wg, Wu, Wd, step_num):
    b, s, d = hidden_states.shape
    x = hidden_states.reshape(-1, d)

    router_logits = x @ gate_w.T
    routing_vals, selected_experts = _topk2(router_logits)
    routing_weights = jax.nn.softmax(routing_vals.astype(jnp.float32), axis=-1)

    ef = selected_experts.reshape(-1).astype(jnp.int32)
    wf = routing_weights.reshape(-1)
    nslots = ef.shape[0]
    onehot = (ef[:, None] == jnp.arange(E, dtype=jnp.int32)[None, :])
    onehot = onehot.astype(jnp.int32)
    csum = jnp.cumsum(onehot, axis=0)
    counts = csum[-1]
    rank = jnp.take_along_axis(csum, ef[:, None], axis=-1)[:, 0] - 1
    ntiles = (counts + TILE - 1) // TILE
    tile_ofs = jnp.concatenate([jnp.zeros((1,), jnp.int32),
                                jnp.cumsum(ntiles)[:-1]])
    pos = tile_ofs[ef] * TILE + rank

    t_padded = jnp.zeros((NP,), jnp.int32).at[pos].set(
        (jnp.arange(nslots, dtype=jnp.int32) // TOP_K))
    w_padded = jnp.zeros((NP,), jnp.float32).at[pos].set(wf)
    tile_expert = jnp.repeat(jnp.arange(E, dtype=jnp.int32), ntiles,
                             total_repeat_length=NT)
    n_real = jnp.sum(ntiles)
    valid = (jnp.arange(NT, dtype=jnp.int32) < n_real).astype(jnp.int32)

    p0 = pos[0::TOP_K]
    p1 = pos[1::TOP_K]

    x_bf = x.astype(jnp.bfloat16)
    x_sorted = x_bf[t_padded]
    out_sorted = _grouped_mlp(x_sorted, Wg, Wu, Wd, w_padded[:, None],
                              tile_expert, valid)
    final = out_sorted[p0] + out_sorted[p1]
    return final.reshape(b, s, d)
